# baseline (speedup 1.0000x reference)
import functools

import jax
import jax.numpy as jnp
from jax import lax
from jax.experimental import pallas as pl
from jax.experimental.pallas import tpu as pltpu
from jax.experimental.pallas import tpu_sc as plsc

N = 10000
F = 128
O = 64
E = 320000

NC, NS = 2, 16
NW = NC * NS

CHUNK = 128
NCHUNK = -(-E // (NW * CHUNK))
EPW = NCHUNK * CHUNK
E_PAD = EPW * NW
R = 10240
RPT = R // NS
DUMMY = N
NCH0 = 37
NCH1 = 2 * NCHUNK - NCH0
DEG_W = 128

_mesh = plsc.VectorSubcoreMesh(core_axis_name="c", subcore_axis_name="s")


@functools.partial(
    pl.kernel,
    out_type=jax.ShapeDtypeStruct((NC, R, DEG_W), jnp.float32),
    mesh=_mesh,
    scratch_types=(
        pltpu.VMEM((CHUNK,), jnp.int32),
        pltpu.VMEM((CHUNK, DEG_W), jnp.float32),
        pltpu.VMEM((CHUNK, DEG_W), jnp.float32),
        pltpu.VMEM_SHARED((R, DEG_W), jnp.float32),
    ),
)
def _seg_deg(dstp, z16, ones_hbm, deg_out, dst_v, ones_v, dstage_v, deg_sh):
    c = lax.axis_index("c")
    s = lax.axis_index("s")
    wid = s * NC + c
    row0 = s * RPT

    pltpu.sync_copy(z16, dstage_v)
    for k in range(RPT // CHUNK):
        pltpu.sync_copy(dstage_v, deg_sh.at[pl.ds(row0 + k * CHUNK, CHUNK)])
    pltpu.sync_copy(ones_hbm, ones_v)
    plsc.subcore_barrier()

    def body(j, carry):
        base = pl.multiple_of(wid * EPW + j * CHUNK, CHUNK)
        pltpu.sync_copy(dstp.at[pl.ds(base, CHUNK)], dst_v)
        pltpu.sync_copy(ones_v, deg_sh.at[dst_v], add=True)
        return carry

    lax.fori_loop(0, NCHUNK, body, 0)
    plsc.subcore_barrier()

    for k in range(RPT // CHUNK):
        r0 = row0 + k * CHUNK
        pltpu.sync_copy(deg_sh.at[pl.ds(r0, CHUNK)], dstage_v)
        pltpu.sync_copy(dstage_v, deg_out.at[c, pl.ds(r0, CHUNK)])


@functools.partial(
    pl.kernel,
    out_type=jax.ShapeDtypeStruct((NC, R, F), jnp.float32),
    mesh=_mesh,
    scratch_types=(
        pltpu.VMEM((CHUNK,), jnp.int32),
        pltpu.VMEM((CHUNK,), jnp.int32),
        pltpu.VMEM((CHUNK, F), jnp.float32),
        pltpu.SemaphoreType.DMA,
        pltpu.VMEM_SHARED((R, F), jnp.float32),
    ),
)
def _seg_sum(y, srcp, dstp, z128, acc_out, src_v, dst_v, rows_v, sem, acc_sh):
    c = lax.axis_index("c")
    s = lax.axis_index("s")
    row0 = s * RPT
    nch = jnp.where(c == 0, NCH0, NCH1)
    chunk0 = c * NS * NCH0 + s * nch

    pltpu.sync_copy(z128, rows_v)
    for k in range(RPT // CHUNK):
        pltpu.sync_copy(rows_v, acc_sh.at[pl.ds(row0 + k * CHUNK, CHUNK)])
    plsc.subcore_barrier()

    def body(j, carry):
        base = pl.multiple_of((chunk0 + j) * CHUNK, CHUNK)
        pltpu.sync_copy(srcp.at[pl.ds(base, CHUNK)], src_v)
        pltpu.sync_copy(dstp.at[pl.ds(base, CHUNK)], dst_v)
        pltpu.async_copy(y.at[src_v], rows_v, sem).wait()
        pltpu.sync_copy(rows_v, acc_sh.at[dst_v], add=True)
        return carry

    lax.fori_loop(0, nch, body, 0)
    plsc.subcore_barrier()

    for k in range(RPT // CHUNK):
        r0 = row0 + k * CHUNK
        pltpu.sync_copy(acc_sh.at[pl.ds(r0, CHUNK)], rows_v)
        pltpu.sync_copy(rows_v, acc_out.at[c, pl.ds(r0, CHUNK)])


BN = 2000
_GRID = (N // BN,)


def _stage_h_body(p0_ref, p1_ref, d0_ref, d1_ref, x_ref, wl_ref, wr_ref,
                  b_ref, h_ref):
    degc = jnp.maximum(d0_ref[:, :1] + d1_ref[:, :1], 1.0)
    agg = (p0_ref[...] + p1_ref[...]) / degc
    pre = (jnp.dot(agg, wl_ref[...], preferred_element_type=jnp.float32)
           + b_ref[...]
           + jnp.dot(x_ref[...], wr_ref[...], preferred_element_type=jnp.float32))
    h_ref[...] = jnp.maximum(pre, 0.0)


def _stage_out_body(q0_ref, q1_ref, d0_ref, d1_ref, h_ref, wl_ref, wr_ref,
                    b_ref, wc_ref, bc_ref, log_ref, emb_ref, soft_ref, hard_ref):
    degc = jnp.maximum(d0_ref[:, :1] + d1_ref[:, :1], 1.0)
    agg = (q0_ref[...] + q1_ref[...]) / degc
    h2 = (jnp.dot(agg, wl_ref[...], preferred_element_type=jnp.float32)
          + b_ref[...]
          + jnp.dot(h_ref[...], wr_ref[...], preferred_element_type=jnp.float32))
    emb_ref[...] = h2
    lg = jnp.dot(h2, wc_ref[...], preferred_element_type=jnp.float32) + bc_ref[...]
    log_ref[...] = lg
    m = jnp.max(lg, axis=1, keepdims=True)
    e = jnp.exp(lg - m)
    soft = e / jnp.sum(e, axis=1, keepdims=True)
    soft_ref[...] = soft
    ms = jnp.max(soft, axis=1, keepdims=True)
    cols = lax.broadcasted_iota(jnp.int32, (BN, O), 1)
    hard_ref[...] = jnp.min(jnp.where(soft == ms, cols, O), axis=1, keepdims=True)


def _rows_spec(w):
    return pl.BlockSpec((BN, w), lambda i: (i, 0))


def _full_spec(a, b):
    return pl.BlockSpec((a, b), lambda i: (0, 0))


_stage_h = pl.pallas_call(
    _stage_h_body,
    grid=_GRID,
    in_specs=[_rows_spec(F), _rows_spec(F), _rows_spec(DEG_W), _rows_spec(DEG_W),
              _rows_spec(F), _full_spec(F, F), _full_spec(F, F), _full_spec(1, F)],
    out_specs=[_rows_spec(F)],
    out_shape=[jax.ShapeDtypeStruct((N, F), jnp.float32)],
)

_stage_out = pl.pallas_call(
    _stage_out_body,
    grid=_GRID,
    in_specs=[_rows_spec(F), _rows_spec(F), _rows_spec(DEG_W), _rows_spec(DEG_W),
              _rows_spec(F), _full_spec(F, F), _full_spec(F, F), _full_spec(1, F),
              _full_spec(F, O), _full_spec(1, O)],
    out_specs=[_rows_spec(O), _rows_spec(F), _rows_spec(O), _rows_spec(1)],
    out_shape=[jax.ShapeDtypeStruct((N, O), jnp.float32),
               jax.ShapeDtypeStruct((N, F), jnp.float32),
               jax.ShapeDtypeStruct((N, O), jnp.float32),
               jax.ShapeDtypeStruct((N, 1), jnp.int32)],
)


def kernel(x, edge_index, W1l, b1l, W1r, W2l, b2l, W2r, Wc, bc):
    src = edge_index[0].astype(jnp.int32)
    dst = edge_index[1].astype(jnp.int32)
    srcp = jnp.concatenate([src, jnp.zeros((E_PAD - E,), jnp.int32)])
    dstp = jnp.concatenate([dst, jnp.full((E_PAD - E,), DUMMY, jnp.int32)])
    srcp, dstp = lax.optimization_barrier((srcp, dstp))

    z128 = jnp.zeros((CHUNK, F), jnp.float32)
    ones = jnp.ones((CHUNK, DEG_W), jnp.float32)
    z128, ones = lax.optimization_barrier((z128, ones))

    acc1 = _seg_sum(x, srcp, dstp, z128)
    degp = _seg_deg(dstp, z128, ones)
    (h,) = _stage_h(acc1[0], acc1[1], degp[0], degp[1], x,
                    W1l.T, W1r.T, b1l.reshape(1, F))
    acc2 = _seg_sum(h, srcp, dstp, z128)
    logits, emb, soft, hard = _stage_out(acc2[0], acc2[1], degp[0], degp[1], h,
                                         W2l.T, W2r.T, b2l.reshape(1, F),
                                         Wc.T, bc.reshape(1, O))
    return logits, emb, soft, hard.reshape(N)


# speedup vs baseline: 1.2232x; 1.2232x over previous
import functools

import jax
import jax.numpy as jnp
from jax import lax
from jax.experimental import pallas as pl
from jax.experimental.pallas import tpu as pltpu
from jax.experimental.pallas import tpu_sc as plsc

N = 10000
F = 128
O = 64
E = 320000

NC, NS = 2, 16
NW = NC * NS

CHUNK = 128
NCHUNK = -(-E // (NW * CHUNK))
EPW = NCHUNK * CHUNK
E_PAD = EPW * NW
R = 10240
RPT = R // NS
DUMMY = N
DEG_W = 128

_mesh = plsc.VectorSubcoreMesh(core_axis_name="c", subcore_axis_name="s")


@functools.partial(
    pl.kernel,
    out_type=jax.ShapeDtypeStruct((NC, R, DEG_W), jnp.float32),
    mesh=_mesh,
    scratch_types=(
        pltpu.VMEM((CHUNK,), jnp.int32),
        pltpu.VMEM((CHUNK, DEG_W), jnp.float32),
        pltpu.VMEM((CHUNK, DEG_W), jnp.float32),
        pltpu.VMEM_SHARED((R, DEG_W), jnp.float32),
    ),
)
def _seg_deg(dstp, z16, ones_hbm, deg_out, dst_v, ones_v, dstage_v, deg_sh):
    c = lax.axis_index("c")
    s = lax.axis_index("s")
    wid = s * NC + c
    row0 = s * RPT

    pltpu.sync_copy(z16, dstage_v)
    for k in range(RPT // CHUNK):
        pltpu.sync_copy(dstage_v, deg_sh.at[pl.ds(row0 + k * CHUNK, CHUNK)])
    pltpu.sync_copy(ones_hbm, ones_v)
    plsc.subcore_barrier()

    def body(j, carry):
        base = pl.multiple_of(wid * EPW + j * CHUNK, CHUNK)
        pltpu.sync_copy(dstp.at[pl.ds(base, CHUNK)], dst_v)
        pltpu.sync_copy(ones_v, deg_sh.at[dst_v], add=True)
        return carry

    lax.fori_loop(0, NCHUNK, body, 0)
    plsc.subcore_barrier()

    for k in range(RPT // CHUNK):
        r0 = row0 + k * CHUNK
        pltpu.sync_copy(deg_sh.at[pl.ds(r0, CHUNK)], dstage_v)
        pltpu.sync_copy(dstage_v, deg_out.at[c, pl.ds(r0, CHUNK)])


@functools.partial(
    pl.kernel,
    out_type=jax.ShapeDtypeStruct((NC, R, F), jnp.float32),
    mesh=_mesh,
    scratch_types=(
        pltpu.VMEM((CHUNK,), jnp.int32),
        pltpu.VMEM((CHUNK,), jnp.int32),
        pltpu.VMEM((CHUNK, F), jnp.float32),
        pltpu.SemaphoreType.DMA,
        pltpu.VMEM_SHARED((R, F), jnp.float32),
    ),
)
def _seg_sum(y, srcp, dstp, z128, acc_out, src_v, dst_v, rows_v, sem, acc_sh):
    c = lax.axis_index("c")
    s = lax.axis_index("s")
    wid = s * NC + c
    row0 = s * RPT

    pltpu.sync_copy(z128, rows_v)
    for k in range(RPT // CHUNK):
        pltpu.sync_copy(rows_v, acc_sh.at[pl.ds(row0 + k * CHUNK, CHUNK)])
    plsc.subcore_barrier()

    def body(j, carry):
        base = pl.multiple_of(wid * EPW + j * CHUNK, CHUNK)
        pltpu.sync_copy(srcp.at[pl.ds(base, CHUNK)], src_v)
        pltpu.sync_copy(dstp.at[pl.ds(base, CHUNK)], dst_v)
        pltpu.async_copy(y.at[src_v], rows_v, sem).wait()
        pltpu.sync_copy(rows_v, acc_sh.at[dst_v], add=True)
        return carry

    lax.fori_loop(0, NCHUNK, body, 0)
    plsc.subcore_barrier()

    for k in range(RPT // CHUNK):
        r0 = row0 + k * CHUNK
        pltpu.sync_copy(acc_sh.at[pl.ds(r0, CHUNK)], rows_v)
        pltpu.sync_copy(rows_v, acc_out.at[c, pl.ds(r0, CHUNK)])


BN = 2000
_GRID = (N // BN,)


def _stage_h_body(p0_ref, p1_ref, d0_ref, d1_ref, x_ref, wl_ref, wr_ref,
                  b_ref, h_ref):
    degc = jnp.maximum(d0_ref[:, :1] + d1_ref[:, :1], 1.0)
    agg = (p0_ref[...] + p1_ref[...]) / degc
    pre = (jnp.dot(agg, wl_ref[...], preferred_element_type=jnp.float32)
           + b_ref[...]
           + jnp.dot(x_ref[...], wr_ref[...], preferred_element_type=jnp.float32))
    h_ref[...] = jnp.maximum(pre, 0.0)


def _stage_out_body(q0_ref, q1_ref, d0_ref, d1_ref, h_ref, wl_ref, wr_ref,
                    b_ref, wc_ref, bc_ref, log_ref, emb_ref, soft_ref, hard_ref):
    degc = jnp.maximum(d0_ref[:, :1] + d1_ref[:, :1], 1.0)
    agg = (q0_ref[...] + q1_ref[...]) / degc
    h2 = (jnp.dot(agg, wl_ref[...], preferred_element_type=jnp.float32)
          + b_ref[...]
          + jnp.dot(h_ref[...], wr_ref[...], preferred_element_type=jnp.float32))
    emb_ref[...] = h2
    lg = jnp.dot(h2, wc_ref[...], preferred_element_type=jnp.float32) + bc_ref[...]
    log_ref[...] = lg
    m = jnp.max(lg, axis=1, keepdims=True)
    e = jnp.exp(lg - m)
    soft = e / jnp.sum(e, axis=1, keepdims=True)
    soft_ref[...] = soft
    ms = jnp.max(soft, axis=1, keepdims=True)
    cols = lax.broadcasted_iota(jnp.int32, (BN, O), 1)
    hard_ref[...] = jnp.min(jnp.where(soft == ms, cols, O), axis=1, keepdims=True)


def _rows_spec(w):
    return pl.BlockSpec((BN, w), lambda i: (i, 0))


def _full_spec(a, b):
    return pl.BlockSpec((a, b), lambda i: (0, 0))


_stage_h = pl.pallas_call(
    _stage_h_body,
    grid=_GRID,
    in_specs=[_rows_spec(F), _rows_spec(F), _rows_spec(DEG_W), _rows_spec(DEG_W),
              _rows_spec(F), _full_spec(F, F), _full_spec(F, F), _full_spec(1, F)],
    out_specs=[_rows_spec(F)],
    out_shape=[jax.ShapeDtypeStruct((N, F), jnp.float32)],
)

_stage_out = pl.pallas_call(
    _stage_out_body,
    grid=_GRID,
    in_specs=[_rows_spec(F), _rows_spec(F), _rows_spec(DEG_W), _rows_spec(DEG_W),
              _rows_spec(F), _full_spec(F, F), _full_spec(F, F), _full_spec(1, F),
              _full_spec(F, O), _full_spec(1, O)],
    out_specs=[_rows_spec(O), _rows_spec(F), _rows_spec(O), _rows_spec(1)],
    out_shape=[jax.ShapeDtypeStruct((N, O), jnp.float32),
               jax.ShapeDtypeStruct((N, F), jnp.float32),
               jax.ShapeDtypeStruct((N, O), jnp.float32),
               jax.ShapeDtypeStruct((N, 1), jnp.int32)],
)


def kernel(x, edge_index, W1l, b1l, W1r, W2l, b2l, W2r, Wc, bc):
    src = edge_index[0].astype(jnp.int32)
    dst = edge_index[1].astype(jnp.int32)
    srcp = jnp.concatenate([src, jnp.zeros((E_PAD - E,), jnp.int32)])
    dstp = jnp.concatenate([dst, jnp.full((E_PAD - E,), DUMMY, jnp.int32)])
    srcp, dstp = lax.optimization_barrier((srcp, dstp))

    z128 = jnp.zeros((CHUNK, F), jnp.float32)
    ones = jnp.ones((CHUNK, DEG_W), jnp.float32)
    z128, ones = lax.optimization_barrier((z128, ones))

    acc1 = _seg_sum(x, srcp, dstp, z128)
    degp = _seg_deg(dstp, z128, ones)
    (h,) = _stage_h(acc1[0], acc1[1], degp[0], degp[1], x,
                    W1l.T, W1r.T, b1l.reshape(1, F))
    acc2 = _seg_sum(h, srcp, dstp, z128)
    logits, emb, soft, hard = _stage_out(acc2[0], acc2[1], degp[0], degp[1], h,
                                         W2l.T, W2r.T, b2l.reshape(1, F),
                                         Wc.T, bc.reshape(1, O))
    return logits, emb, soft, hard.reshape(N)
